# Initial kernel scaffold; baseline (speedup 1.0000x reference)
import jax
import jax.numpy as jnp
from jax.experimental import pallas as pl


def kernel(points, drag_point, drag_vector, joint_type, joint_axis, joint_origin, params):
    raise NotImplementedError("write your pallas kernel here")


# speedup vs baseline: 2.3795x; 2.3795x over previous
import jax
import jax.numpy as jnp
from jax import lax
from jax.experimental import pallas as pl
from jax.experimental.pallas import tpu as pltpu
from jax.experimental.pallas import tpu_sc as plsc

_EPS = 1e-5


def _merge16(ka, va, kb, vb):
  kbr = lax.rev(kb, (0,))
  vbr = lax.rev(vb, (0,))
  m = ka <= kbr
  klo = jnp.where(m, ka, kbr)
  vlo = jnp.where(m, va, vbr)
  khi = jnp.where(m, kbr, ka)
  vhi = jnp.where(m, vbr, va)
  klo, vlo = plsc.sort_key_val(klo, vlo)
  khi, vhi = plsc.sort_key_val(khi, vhi)
  return klo, vlo, khi, vhi


def _sc_topk_kernel(xs_hbm, ys_hbm, zs_hbm, cx_hbm, cy_hbm, cz_hbm, out_hbm,
                    xv, yv, zv, cv, iv, sem):
  n = xs_hbm.shape[1]
  nchunks = n // 16
  wid = lax.axis_index("s") * 2 + lax.axis_index("c")
  pltpu.sync_copy(xs_hbm.at[wid], xv)
  pltpu.sync_copy(ys_hbm.at[wid], yv)
  pltpu.sync_copy(zs_hbm.at[wid], zv)
  pltpu.sync_copy(cx_hbm.at[wid], cv.at[0])
  pltpu.sync_copy(cy_hbm.at[wid], cv.at[1])
  pltpu.sync_copy(cz_hbm.at[wid], cv.at[2])
  cx = cv[0]
  cy = cv[1]
  cz = cv[2]

  def dist(t):
    dx = xv[pl.ds(t * 16, 16)] - cx
    dy = yv[pl.ds(t * 16, 16)] - cy
    dz = zv[pl.ds(t * 16, 16)] - cz
    idx = lax.iota(jnp.int32, 16) + t * 16
    return dx * dx + dy * dy + dz * dz, idx

  d0, i0 = dist(0)
  d1, i1 = dist(1)
  d0, i0 = plsc.sort_key_val(d0, i0)
  d1, i1 = plsc.sort_key_val(d1, i1)
  t0, j0, t1, j1 = _merge16(d0, i0, d1, i1)

  def body(t, carry):
    t0, j0, t1, j1 = carry
    d, i = dist(t)
    thr = jnp.max(t1)

    def merge(args):
      t0, j0, t1, j1, d, i = args
      ds, isrt = plsc.sort_key_val(d, i)
      dr = lax.rev(ds, (0,))
      ir = lax.rev(isrt, (0,))
      m = t1 <= dr
      ck = jnp.where(m, t1, dr)
      cv_ = jnp.where(m, j1, ir)
      ck, cv_ = plsc.sort_key_val(ck, cv_)
      return _merge16(t0, j0, ck, cv_)

    def skip(args):
      t0, j0, t1, j1, _, _ = args
      return t0, j0, t1, j1

    any_hit = jnp.any(d < thr)
    return lax.cond(any_hit, merge, skip, (t0, j0, t1, j1, d, i))

  t0, j0, t1, j1 = lax.fori_loop(2, nchunks, body, (t0, j0, t1, j1))
  iv[pl.ds(0, 16)] = j0
  iv[pl.ds(16, 16)] = j1
  pltpu.sync_copy(iv, out_hbm.at[wid])


@jax.jit
def _sc_topk_call(xs, ys, zs, cx, cy, cz):
  nrows = xs.shape[0]
  mesh = plsc.VectorSubcoreMesh(core_axis_name="c", subcore_axis_name="s")
  kfn = pl.kernel(
      _sc_topk_kernel,
      out_type=jax.ShapeDtypeStruct((nrows, 32), jnp.int32),
      mesh=mesh,
      compiler_params=pltpu.CompilerParams(needs_layout_passes=False),
      scratch_types=[
          pltpu.VMEM((xs.shape[1],), jnp.float32),
          pltpu.VMEM((xs.shape[1],), jnp.float32),
          pltpu.VMEM((xs.shape[1],), jnp.float32),
          pltpu.VMEM((3, 16), jnp.float32),
          pltpu.VMEM((32,), jnp.int32),
          pltpu.SemaphoreType.DMA,
      ],
  )
  return kfn(xs, ys, zs, cx, cy, cz)


def _group_affine(h, gmat, gamma, beta, count):
  s = jnp.sum(h, axis=0, keepdims=True)
  sq = jnp.sum(h * h, axis=0, keepdims=True)
  sg = lax.dot_general(s, gmat, (((1,), (0,)), ((), ())))
  sqg = lax.dot_general(sq, gmat, (((1,), (0,)), ((), ())))
  mean_g = sg / count
  var_g = sqg / count - mean_g * mean_g
  inv_g = lax.rsqrt(var_g + _EPS)
  mean = lax.dot_general(mean_g, gmat, (((1,), (1,)), ((), ())))
  inv = lax.dot_general(inv_g, gmat, (((1,), (1,)), ((), ())))
  a = inv * gamma
  d = beta - mean * a
  return a, d


def _indicator(c, g):
  per = c // g
  ci = lax.broadcasted_iota(jnp.int32, (c, g), 0)
  gi = lax.broadcasted_iota(jnp.int32, (c, g), 1)
  return (ci // per == gi).astype(jnp.float32)


def _encoder_kernel(x_ref, idx_ref, w1, b1, g1, be1, w2, b2, g2, be2,
                    w3, b3, g3, be3, gmax_ref, jl_ref, dl_ref):
  n = x_ref.shape[1]
  x = x_ref[0]
  h1 = lax.dot_general(x, w1[...], (((1,), (1,)), ((), ()))) + b1[...]
  a1, d1 = _group_affine(h1, _indicator(128, 16), g1[...], be1[...],
                         float(n * 8))
  h1 = jnp.maximum(h1 * a1 + d1, 0.0)
  h2 = lax.dot_general(h1, w2[...], (((1,), (1,)), ((), ()))) + b2[...]
  a2, d2 = _group_affine(h2, _indicator(256, 32), g2[...], be2[...],
                         float(n * 8))
  h2 = jnp.maximum(h2 * a2 + d2, 0.0)
  h3 = lax.dot_general(h2, w3[...], (((1,), (1,)), ((), ()))) + b3[...]
  a3, d3 = _group_affine(h3, _indicator(1024, 64), g3[...], be3[...],
                         float(n * 16))
  hmax = jnp.max(h3, axis=0, keepdims=True)
  hmin = jnp.min(h3, axis=0, keepdims=True)
  pooled = jnp.where(a3 >= 0.0, a3 * hmax, a3 * hmin) + d3
  gmax_ref[0] = jnp.maximum(pooled, 0.0)
  idx = idx_ref[0]
  ni = lax.broadcasted_iota(jnp.int32, (n, 64), 0)
  onehot = (ni == idx).astype(jnp.float32)
  rows2 = lax.dot_general(onehot, h2, (((0,), (0,)), ((), ())))
  rows3 = lax.dot_general(rows2, w3[...], (((1,), (1,)), ((), ()))) + b3[...]
  pf64 = jnp.maximum(rows3 * a3 + d3, 0.0)
  jl_ref[0] = jnp.max(pf64[0:32], axis=0, keepdims=True)
  dl_ref[0] = jnp.max(pf64[32:64], axis=0, keepdims=True)


@jax.jit
def _encoder_call(points, idx64, p):
  b, n, _ = points.shape
  row = lambda i: pl.BlockSpec((1, n, 4), lambda j: (j, 0, 0))
  full = lambda s: pl.BlockSpec(s, lambda j: tuple(0 for _ in s))
  out = pl.BlockSpec((1, 1, 1024), lambda j: (j, 0, 0))
  specs = [
      pl.BlockSpec((1, n, 4), lambda j: (j, 0, 0)),
      pl.BlockSpec((1, 1, 64), lambda j: (j, 0, 0)),
      full((128, 4)), full((1, 128)), full((1, 128)), full((1, 128)),
      full((256, 128)), full((1, 256)), full((1, 256)), full((1, 256)),
      full((1024, 256)), full((1, 1024)), full((1, 1024)), full((1, 1024)),
  ]
  r2 = lambda a: a.reshape(1, -1)
  args = (points, idx64,
          p['pc_w1'], r2(p['pc_b1']), r2(p['pc_g1']), r2(p['pc_be1']),
          p['pc_w2'], r2(p['pc_b2']), r2(p['pc_g2']), r2(p['pc_be2']),
          p['pc_w3'], r2(p['pc_b3']), r2(p['pc_g3']), r2(p['pc_be3']))
  return pl.pallas_call(
      _encoder_kernel,
      grid=(b,),
      in_specs=specs,
      out_specs=[out, out, out],
      out_shape=[jax.ShapeDtypeStruct((b, 1, 1024), jnp.float32)] * 3,
  )(*args)


def _mm(x, w):
  return lax.dot_general(x, w, (((1,), (1,)), ((), ())))


def _lnorm(x, g, b):
  m = jnp.mean(x, axis=1, keepdims=True)
  v = jnp.mean(x * x, axis=1, keepdims=True) - m * m
  return (x - m) * lax.rsqrt(v + _EPS) * g + b


def _head_kernel(gmax, jl, dl, dp, dv, jt, ja, jo, refs, out_ref):
  r = lambda k: refs[k][...]
  gm = gmax[...]
  g = _mm(gm, r('pc_w4')) + r('pc_b4')
  g = jnp.maximum(_lnorm(g, r('pc_ln4g'), r('pc_ln4b')), 0.0)
  g = _mm(g, r('pc_w5')) + r('pc_b5')

  dpv = dp[...]
  dvv = dv[...]
  jov = jo[...]
  di = jnp.concatenate([dpv, dvv], axis=1)
  df = _mm(_lnorm(jnp.maximum(_mm(di, r('de_w1')) + r('de_b1'), 0.0),
                  r('de_lng'), r('de_lnb')), r('de_w2')) + r('de_b2')
  rel = dpv - jov
  rf = _mm(_lnorm(jnp.maximum(_mm(rel, r('rp_w1')) + r('rp_b1'), 0.0),
                  r('rp_lng'), r('rp_lnb')), r('rp_w2')) + r('rp_b2')
  mag = jnp.sqrt(jnp.sum(dvv * dvv, axis=1, keepdims=True))
  mf = _mm(jnp.maximum(_mm(mag, r('mg_w1')) + r('mg_b1'), 0.0),
           r('mg_w2')) + r('mg_b2')
  comb = jnp.concatenate([df, rf, mf], axis=1)
  drag_feat = _mm(jnp.maximum(_mm(comb, r('df_w1')) + r('df_b1'), 0.0),
                  r('df_w2')) + r('df_b2')

  onehot = (jt[...] == lax.broadcasted_iota(jnp.int32, (jt.shape[0], 2), 1))
  tf = lax.dot_general(onehot.astype(jnp.float32), r('emb'),
                       (((1,), (0,)), ((), ())))
  af = _mm(jnp.maximum(_mm(ja[...], r('ax_w1')) + r('ax_b1'), 0.0),
           r('ax_w2')) + r('ax_b2')
  of = _mm(jnp.maximum(_mm(jov, r('or_w1')) + r('or_b1'), 0.0),
           r('or_w2')) + r('or_b2')
  jc = jnp.concatenate([tf, af, of], axis=1)
  joint_feat = _mm(jnp.maximum(_mm(jc, r('jf_w1')) + r('jf_b1'), 0.0),
                   r('jf_w2')) + r('jf_b2')

  jlf = _mm(jnp.maximum(_mm(jl[...], r('jm_w1')) + r('jm_b1'), 0.0),
            r('jm_w2')) + r('jm_b2')
  dlf = _mm(jnp.maximum(_mm(dl[...], r('dm_w1')) + r('dm_b1'), 0.0),
            r('dm_w2')) + r('dm_b2')
  loc = jnp.concatenate([jlf, dlf], axis=1)
  local = _mm(jnp.maximum(_mm(loc, r('lf_w1')) + r('lf_b1'), 0.0),
              r('lf_w2')) + r('lf_b2')
  local = (_mm(joint_feat, r('fs_w')) + r('fs_b')) * local + \
          _mm(joint_feat, r('fsh_w')) + r('fsh_b')
  vi = jnp.concatenate([local, joint_feat, drag_feat], axis=1)
  mu = _mm(vi, r('mu_w')) + r('mu_b')
  lv = _mm(vi, r('lv_w')) + r('lv_b')
  out_ref[...] = jnp.concatenate([mu, lv, g], axis=1)


_HEAD_KEYS = (
    'pc_w4', 'pc_b4', 'pc_ln4g', 'pc_ln4b', 'pc_w5', 'pc_b5',
    'de_w1', 'de_b1', 'de_lng', 'de_lnb', 'de_w2', 'de_b2',
    'rp_w1', 'rp_b1', 'rp_lng', 'rp_lnb', 'rp_w2', 'rp_b2',
    'mg_w1', 'mg_b1', 'mg_w2', 'mg_b2',
    'df_w1', 'df_b1', 'df_w2', 'df_b2',
    'emb',
    'ax_w1', 'ax_b1', 'ax_w2', 'ax_b2',
    'or_w1', 'or_b1', 'or_w2', 'or_b2',
    'jf_w1', 'jf_b1', 'jf_w2', 'jf_b2',
    'jm_w1', 'jm_b1', 'jm_w2', 'jm_b2',
    'dm_w1', 'dm_b1', 'dm_w2', 'dm_b2',
    'lf_w1', 'lf_b1', 'lf_w2', 'lf_b2',
    'fs_w', 'fs_b', 'fsh_w', 'fsh_b',
    'mu_w', 'mu_b', 'lv_w', 'lv_b',
)


@jax.jit
def _head_call(gmax, jl, dl, dp, dv, jt, ja, jo, p):
  b = gmax.shape[0]
  refs = {}
  for k in _HEAD_KEYS:
    a = p[k]
    refs[k] = a.reshape(1, -1) if a.ndim == 1 else a
  return pl.pallas_call(
      _head_kernel,
      out_shape=jax.ShapeDtypeStruct((b, 2048), jnp.float32),
  )(gmax, jl, dl, dp, dv, jt.reshape(b, 1).astype(jnp.int32), ja, jo, refs)


def kernel(points, drag_point, drag_vector, joint_type, joint_axis,
           joint_origin, params):
  b, n, _ = points.shape
  xs = jnp.concatenate([points[:, :, 0], points[:, :, 0]], axis=0)
  ys = jnp.concatenate([points[:, :, 1], points[:, :, 1]], axis=0)
  zs = jnp.concatenate([points[:, :, 2], points[:, :, 2]], axis=0)
  centers = jnp.concatenate([joint_origin, drag_point], axis=0)
  cx = jnp.broadcast_to(centers[:, 0:1], (2 * b, 16))
  cy = jnp.broadcast_to(centers[:, 1:2], (2 * b, 16))
  cz = jnp.broadcast_to(centers[:, 2:3], (2 * b, 16))
  idx = _sc_topk_call(xs, ys, zs, cx, cy, cz)
  idx64 = jnp.concatenate([idx[:b], idx[b:]], axis=1)
  gmax, jl, dl = _encoder_call(points, idx64.reshape(b, 1, 64), params)
  gmax = gmax.reshape(b, 1024)
  jl = jl.reshape(b, 1024)
  dl = dl.reshape(b, 1024)
  return _head_call(gmax, jl, dl, drag_point, drag_vector, joint_type,
                    joint_axis, joint_origin, params)
